# Initial kernel scaffold; baseline (speedup 1.0000x reference)
import jax
import jax.numpy as jnp
from jax.experimental import pallas as pl


def kernel(x, edge_index, valid_actions, current_node, current_partial_path, Wn, bn, W0, as0, ad0, b0, W1, as1, ad1, b1, pos_emb, Wc, bc, Ws1, bs1, Ws2, bs2):
    raise NotImplementedError("write your pallas kernel here")


# speedup vs baseline: 24.1119x; 24.1119x over previous
import functools

import jax
import jax.numpy as jnp
from jax import lax
from jax.experimental import pallas as pl
from jax.experimental.pallas import tpu as pltpu
from jax.experimental.pallas import tpu_sc as plsc

N_NODES = 10000
D_FEAT = 128
HID = 64
WROW = 128
SCALE_G = 5
NC, NS, LANES = 2, 16, 16
NW = NC * NS
CHUNK = 128
E_EDGES = 320000
E_TOT = E_EDGES + N_NODES
CH_PER_TILE = -(-E_TOT // (NW * CHUNK))
E_PAD = NW * CH_PER_TILE * CHUNK
NPACK = E_PAD // CHUNK
NEG_SLOPE = 0.2
NPAD = 10240
TSTRIPE = NPAD // NS
NWB = TSTRIPE // CHUNK
NACT = 256
PLEN = 20
NIDX = 288


_BN = 1000


def _prep0_body(x_ref, wn_ref, bn_ref, w0_ref, as_ref, ad_ref,
                hwa_ref, asrc_ref, adst_ref):
    t = jnp.dot(x_ref[...], wn_ref[...], preferred_element_type=jnp.float32)
    t = t + bn_ref[...][None, :]
    hw = jnp.dot(t, w0_ref[...], preferred_element_type=jnp.float32)
    hwa_ref[...] = jnp.concatenate(
        [hw, jnp.ones((_BN, 1), jnp.float32),
         jnp.zeros((_BN, WROW - HID - 1), jnp.float32)], axis=1)
    asrc_ref[...] = jnp.sum(hw * as_ref[...][None, :], axis=1).reshape(
        1, 1, _BN)
    adst_ref[...] = jnp.sum(hw * ad_ref[...][None, :], axis=1).reshape(
        1, 1, _BN)


def _prep1_body(parts_ref, b_ref, w_ref, as_ref, ad_ref,
                hwa_ref, asrc_ref, adst_ref):
    p = parts_ref[0] + parts_ref[1]
    den = p[:, HID:HID + 1] + 1e-16
    h = jnp.maximum(p[:, :HID] / den + b_ref[...][None, :], 0.0)
    hw = jnp.dot(h, w_ref[...], preferred_element_type=jnp.float32)
    hwa_ref[...] = jnp.concatenate(
        [hw, jnp.ones((_BN, 1), jnp.float32),
         jnp.zeros((_BN, WROW - HID - 1), jnp.float32)], axis=1)
    asrc_ref[...] = jnp.sum(hw * as_ref[...][None, :], axis=1).reshape(
        1, 1, _BN)
    adst_ref[...] = jnp.sum(hw * ad_ref[...][None, :], axis=1).reshape(
        1, 1, _BN)


def _prep0(x, Wn, bn, W0, as0, ad0):
    g = N_NODES // _BN
    return pl.pallas_call(
        _prep0_body,
        grid=(g,),
        in_specs=[
            pl.BlockSpec((_BN, D_FEAT), lambda i: (i, 0)),
            pl.BlockSpec((D_FEAT, HID), lambda i: (0, 0)),
            pl.BlockSpec((HID,), lambda i: (0,)),
            pl.BlockSpec((HID, HID), lambda i: (0, 0)),
            pl.BlockSpec((HID,), lambda i: (0,)),
            pl.BlockSpec((HID,), lambda i: (0,)),
        ],
        out_specs=[
            pl.BlockSpec((_BN, WROW), lambda i: (i, 0)),
            pl.BlockSpec((1, 1, _BN), lambda i: (i, 0, 0)),
            pl.BlockSpec((1, 1, _BN), lambda i: (i, 0, 0)),
        ],
        out_shape=[
            jax.ShapeDtypeStruct((N_NODES, WROW), jnp.float32),
            jax.ShapeDtypeStruct((N_NODES // _BN, 1, _BN), jnp.float32),
            jax.ShapeDtypeStruct((N_NODES // _BN, 1, _BN), jnp.float32),
        ],
    )(x, Wn, bn, W0, as0, ad0)


def _prep1(parts, b0, W1, as1, ad1):
    g = N_NODES // _BN
    return pl.pallas_call(
        _prep1_body,
        grid=(g,),
        in_specs=[
            pl.BlockSpec((2, _BN, WROW), lambda i: (0, i, 0)),
            pl.BlockSpec((HID,), lambda i: (0,)),
            pl.BlockSpec((HID, HID), lambda i: (0, 0)),
            pl.BlockSpec((HID,), lambda i: (0,)),
            pl.BlockSpec((HID,), lambda i: (0,)),
        ],
        out_specs=[
            pl.BlockSpec((_BN, WROW), lambda i: (i, 0)),
            pl.BlockSpec((1, 1, _BN), lambda i: (i, 0, 0)),
            pl.BlockSpec((1, 1, _BN), lambda i: (i, 0, 0)),
        ],
        out_shape=[
            jax.ShapeDtypeStruct((N_NODES, WROW), jnp.float32),
            jax.ShapeDtypeStruct((N_NODES // _BN, 1, _BN), jnp.float32),
            jax.ShapeDtypeStruct((N_NODES // _BN, 1, _BN), jnp.float32),
        ],
    )(parts, b0, W1, as1, ad1)


def _sc_edge_body(hwa_hbm, src_hbm, dst_hbm, asrc_hbm, adst_hbm, out_hbm,
                  asrc_v, adst_v, sidx_v, didx_v, rows_v, ex_v, num_sh,
                  sem_g, sem_s):
    cid = lax.axis_index("c")
    sid = lax.axis_index("s")
    wid = cid * NS + sid

    zero16 = jnp.zeros((LANES,), jnp.float32)

    def _zrow(i, _):
        for g in range(WROW // LANES):
            rows_v[0, i, g * LANES:(g + 1) * LANES] = zero16
        return 0
    lax.fori_loop(0, CHUNK, _zrow, 0)
    base = sid * TSTRIPE
    for piece in range(NWB):
        pltpu.sync_copy(rows_v.at[0],
                        num_sh.at[pl.ds(base + piece * CHUNK, CHUNK)])

    for i in range(N_NODES // 1000):
        pltpu.sync_copy(asrc_hbm.at[i, 0], asrc_v.at[pl.ds(i * 1000, 1000)])
        pltpu.sync_copy(adst_hbm.at[i, 0], adst_v.at[pl.ds(i * 1000, 1000)])

    ninf = jnp.full((LANES,), -jnp.inf, jnp.float32)

    def _mx(i, carry):
        ms, md = carry
        return (jnp.maximum(ms, asrc_v[pl.ds(i * LANES, LANES)]),
                jnp.maximum(md, adst_v[pl.ds(i * LANES, LANES)]))
    ms, md = lax.fori_loop(0, N_NODES // LANES, _mx, (ninf, ninf))
    ex_v[0:LANES] = ms
    ex_v[LANES:2 * LANES] = md
    msx, mdx = ninf, ninf
    for j in range(LANES):
        jv = jnp.full((LANES,), j, jnp.int32)
        msx = jnp.maximum(msx, plsc.load_gather(ex_v, [jv]))
        mdx = jnp.maximum(mdx, plsc.load_gather(ex_v, [jv + LANES]))
    cv = msx + mdx
    c16 = jnp.where(cv >= 0, cv, NEG_SLOPE * cv)

    plsc.subcore_barrier()

    iot = lax.iota(jnp.int32, LANES)

    def _chunk(j, _):
        row = wid * CH_PER_TILE + j
        pltpu.sync_copy(src_hbm.at[row], sidx_v)
        pltpu.sync_copy(dst_hbm.at[row], didx_v)
        pltpu.async_copy(hwa_hbm.at[sidx_v.at[0]], rows_v.at[0], sem_g).wait()
        for g in range(CHUNK // LANES):
            sl = pl.ds(g * LANES, LANES)
            s16 = sidx_v[0, sl]
            d16 = didx_v[0, sl]
            a = plsc.load_gather(asrc_v, [s16]) + plsc.load_gather(adst_v, [d16])
            al = jnp.where(a >= 0, a, NEG_SLOPE * a)
            ex = jnp.exp(al - c16)
            pos = jnp.full((LANES,), row * CHUNK + g * LANES, jnp.int32) + iot
            ex_v[sl] = jnp.where(pos < E_TOT, ex, 0.0)

        def _scale(k, _):
            kv = jnp.full((LANES,), k, jnp.int32)
            ev = plsc.load_gather(ex_v, [kv])
            for g in range(SCALE_G):
                sl = pl.ds(g * LANES, LANES)
                rows_v[0, k, sl] = rows_v[0, k, sl] * ev
            return 0
        lax.fori_loop(0, CHUNK, _scale, 0)
        pltpu.async_copy(rows_v.at[0], num_sh.at[didx_v.at[0]],
                         sem_s, add=True).wait()
        return 0
    lax.fori_loop(0, CH_PER_TILE, _chunk, 0)

    plsc.subcore_barrier()
    for piece in range(NWB):
        off = base + piece * CHUNK
        pltpu.sync_copy(num_sh.at[pl.ds(off, CHUNK)],
                        out_hbm.at[cid, pl.ds(off, CHUNK)])


def _sc_edge(hwa, src2d, dst2d, asrc, adst):
    mesh = plsc.VectorSubcoreMesh(core_axis_name="c", subcore_axis_name="s")
    return pl.kernel(
        _sc_edge_body,
        out_type=jax.ShapeDtypeStruct((NC, NPAD, WROW), jnp.float32),
        mesh=mesh,
        compiler_params=pltpu.CompilerParams(needs_layout_passes=False),
        scratch_types=[
            pltpu.VMEM((N_NODES,), jnp.float32),
            pltpu.VMEM((N_NODES,), jnp.float32),
            pltpu.VMEM((1, CHUNK), jnp.int32),
            pltpu.VMEM((1, CHUNK), jnp.int32),
            pltpu.VMEM((1, CHUNK, WROW), jnp.float32),
            pltpu.VMEM((CHUNK,), jnp.float32),
            pltpu.VMEM_SHARED((NPAD, WROW), jnp.float32),
            pltpu.SemaphoreType.DMA,
            pltpu.SemaphoreType.DMA,
        ],
    )(hwa, src2d, dst2d, asrc, adst)


def _head_body(parts_ref, idx_ref, b1_ref, pe_ref, wc_ref, bc_ref,
               ws1_ref, bs1_ref, ws2_ref, bs2_ref, out_ref):
    p = parts_ref[0] + parts_ref[1]
    idx = idx_ref[...].reshape(NIDX, 1)
    node_iota = lax.broadcasted_iota(jnp.int32, (NIDX, NPAD), 1)
    oh = (idx == node_iota).astype(jnp.float32)
    rows = jnp.dot(oh, p, preferred_element_type=jnp.float32)
    den = rows[:, HID:HID + 1] + 1e-16
    h2 = jnp.maximum(rows[:, :HID] / den + b1_ref[...][None, :], 0.0)
    hp = h2[NACT:NACT + PLEN]
    wv = 0.5 + lax.broadcasted_iota(jnp.int32, (PLEN, 1), 0).astype(
        jnp.float32) * (0.5 / 19.0)
    combined = hp + pe_ref[...]
    ws = jnp.sum(wv * combined, axis=0, keepdims=True) / 15.0
    path_emb = jnp.dot(ws, wc_ref[...],
                       preferred_element_type=jnp.float32) + bc_ref[...][None, :]
    curv = h2[NACT + PLEN:NACT + PLEN + 1]
    basev = (jnp.dot(curv, ws1_ref[0:HID, :],
                     preferred_element_type=jnp.float32)
             + jnp.dot(path_emb, ws1_ref[2 * HID:3 * HID, :],
                       preferred_element_type=jnp.float32)
             + bs1_ref[...][None, :])
    act = jnp.dot(h2[:NACT], ws1_ref[HID:2 * HID, :],
                  preferred_element_type=jnp.float32)
    pre = jnp.dot(jnp.maximum(act + basev, 0.0), ws2_ref[...],
                  preferred_element_type=jnp.float32) + bs2_ref[...][None, :]
    s = pre.reshape(1, NACT)
    m = jnp.max(s)
    e = jnp.exp(s - m)
    out_ref[...] = (e / jnp.sum(e)).reshape(NACT)


def _head(parts, idx_all, b1, pos_emb, Wc, bc, Ws1, bs1, Ws2, bs2):
    return pl.pallas_call(
        _head_body,
        grid=(1,),
        in_specs=[
            pl.BlockSpec((2, NPAD, WROW), lambda i: (0, 0, 0)),
            pl.BlockSpec((NIDX,), lambda i: (0,)),
            pl.BlockSpec((HID,), lambda i: (0,)),
            pl.BlockSpec((PLEN, HID), lambda i: (0, 0)),
            pl.BlockSpec((HID, HID), lambda i: (0, 0)),
            pl.BlockSpec((HID,), lambda i: (0,)),
            pl.BlockSpec((4 * HID, HID), lambda i: (0, 0)),
            pl.BlockSpec((HID,), lambda i: (0,)),
            pl.BlockSpec((HID, 1), lambda i: (0, 0)),
            pl.BlockSpec((1,), lambda i: (0,)),
        ],
        out_specs=pl.BlockSpec((NACT,), lambda i: (0,)),
        out_shape=jax.ShapeDtypeStruct((NACT,), jnp.float32),
    )(parts, idx_all, b1, pos_emb, Wc, bc, Ws1, bs1, Ws2, bs2)


def kernel(x, edge_index, valid_actions, current_node, current_partial_path,
           Wn, bn, W0, as0, ad0, b0, W1, as1, ad1, b1,
           pos_emb, Wc, bc, Ws1, bs1, Ws2, bs2):
    cur = jnp.asarray(current_node, jnp.int32)
    loop = jnp.arange(N_NODES, dtype=jnp.int32)
    src = jnp.concatenate([edge_index[0], loop])
    dst = jnp.concatenate([edge_index[1], loop])
    pad = E_PAD - E_TOT
    src2d = jnp.pad(src, (0, pad)).reshape(NPACK, 1, CHUNK)
    dst2d = jnp.pad(dst, (0, pad)).reshape(NPACK, 1, CHUNK)

    hwa0, asrc0, adst0 = _prep0(x, Wn, bn, W0, as0, ad0)
    parts0 = _sc_edge(hwa0, src2d, dst2d, asrc0, adst0)
    hwa1, asrc1, adst1 = _prep1(parts0, b0, W1, as1, ad1)
    parts1 = _sc_edge(hwa1, src2d, dst2d, asrc1, adst1)

    idx_all = jnp.concatenate([
        valid_actions.astype(jnp.int32),
        current_partial_path.astype(jnp.int32),
        cur[None],
        jnp.zeros((NIDX - NACT - PLEN - 1,), jnp.int32),
    ])
    return _head(parts1, idx_all, b1, pos_emb, Wc, bc, Ws1, bs1, Ws2, bs2)
